# baseline (speedup 1.0000x reference)
import jax
import jax.numpy as jnp
from jax import lax
from jax.experimental import pallas as pl
from jax.experimental.pallas import tpu as pltpu
from jax.experimental.pallas import tpu_sc as plsc

_NC, _NS = 2, 16
_NW = _NC * _NS

_SC_ROWS = 41600
_CP_CHUNK = 320
_CP_NCHUNK = _SC_ROWS // 2 // _CP_CHUNK
_CP_NBUF = 4
_TC_BLK = 800
_TC_N = (100000 - _SC_ROWS) // _TC_BLK
_TC_OFF = _SC_ROWS // _TC_BLK


def _sc_gather_body(table_hbm, idx_hbm, out_hbm, idx_v, rows_v, sem):
    bpw = idx_v.shape[0]
    wid = lax.axis_index("s") * _NC + lax.axis_index("c")
    base = wid * bpw
    pltpu.sync_copy(idx_hbm.at[pl.ds(base, bpw)], idx_v)
    pltpu.async_copy(table_hbm.at[idx_v], rows_v, sem).wait()
    pltpu.sync_copy(rows_v, out_hbm.at[pl.ds(base, bpw)])


def _sc_copy_body(src_hbm, dst_hbm, buf0, buf1, buf2, buf3,
                  sem_r0, sem_r1, sem_r2, sem_r3,
                  sem_w0, sem_w1, sem_w2, sem_w3):
    core = lax.axis_index("c")
    sid = lax.axis_index("s")
    base = core * (_CP_NCHUNK * _CP_CHUNK)
    bufs = (buf0, buf1, buf2, buf3)
    sems_r = (sem_r0, sem_r1, sem_r2, sem_r3)
    sems_w = (sem_w0, sem_w1, sem_w2, sem_w3)

    def rd(k, p):
        return pltpu.make_async_copy(
            src_hbm.at[pl.ds(base + k * _CP_CHUNK, _CP_CHUNK)], bufs[p],
            sems_r[p])

    def wr(k, p):
        return pltpu.make_async_copy(
            bufs[p], dst_hbm.at[pl.ds(base + k * _CP_CHUNK, _CP_CHUNK)],
            sems_w[p])

    def phased(p, fn):
        for i in range(_CP_NBUF):
            @pl.when(p == i)
            def _(i=i):
                fn(i)

    @pl.when(sid == 0)
    def _():
        for j in range(_CP_NBUF - 1):
            rd(j, j).start()

        def step(k, _):
            def consume(pp):
                rd(k, pp).wait()
                wr(k, pp).start()

            phased(lax.rem(k, _CP_NBUF), consume)

            @pl.when(k + _CP_NBUF - 1 < _CP_NCHUNK)
            def _():
                def prefetch(pp):
                    @pl.when(k >= 1)
                    def _():
                        wr(k - 1, pp).wait()
                    rd(k + _CP_NBUF - 1, pp).start()

                phased(lax.rem(k + _CP_NBUF - 1, _CP_NBUF), prefetch)

            return 0

        lax.fori_loop(0, _CP_NCHUNK, step, 0)
        for k in range(_CP_NCHUNK - _CP_NBUF, _CP_NCHUNK):
            wr(k, k % _CP_NBUF).wait()


def _mm_body(head_ref, g_ref, wt_ref, bt_ref, wg_ref, bg_ref,
             tok_ref, graph_ref):
    g = g_ref[...]
    t = lax.dot_general(g, wt_ref[...], (((1,), (1,)), ((), ())),
                        preferred_element_type=jnp.float32)
    t = t + bt_ref[...][None, :]
    h = lax.dot_general(g, wg_ref[...], (((1,), (1,)), ((), ())),
                        preferred_element_type=jnp.float32)
    h = h + bg_ref[...][None, :]
    tok_ref[:, 0:1, :] = t[:, None, :]
    tok_ref[:, 1:, :] = head_ref[:, 1:, :]
    graph_ref[...] = h


def _tc_copy_body(cp_any, src_ref, out_ref):
    del cp_any
    out_ref[...] = src_ref[...]


def _scatter_body(idx_ref, g_ref, cp_ref, out_ref, sem):
    del cp_ref
    n = g_ref.shape[0]

    def row(i):
        return pltpu.make_async_copy(
            g_ref.at[pl.ds(i, 1)],
            out_ref.at[pl.ds(idx_ref[i], 1)],
            sem,
        )

    def fire(i, _):
        row(i).start()
        return 0

    def drain(i, _):
        row(i).wait()
        return 0

    lax.fori_loop(0, n, fire, 0)
    lax.fori_loop(0, n, drain, 0)


def kernel(text, all_user_feature, user_neighbor_index,
           W_text, b_text, W_graph, b_graph):
    B, S, D = text.shape
    M = all_user_feature.shape[0]
    user_index = user_neighbor_index[:, 0]

    bpw = B // _NW
    graph_ini = pl.kernel(
        _sc_gather_body,
        out_type=jax.ShapeDtypeStruct((B, D), jnp.float32),
        mesh=plsc.VectorSubcoreMesh(core_axis_name="c", subcore_axis_name="s"),
        scratch_types=[
            pltpu.VMEM((bpw,), jnp.int32),
            pltpu.VMEM((bpw, D), jnp.float32),
            pltpu.SemaphoreType.DMA,
        ],
    )(all_user_feature, user_index)

    text_out, graph = pl.pallas_call(
        _mm_body,
        grid=(1,),
        in_specs=[
            pl.BlockSpec((B, 8, D), lambda i: (0, 0, 0)),
            pl.BlockSpec((B, D), lambda i: (0, 0)),
            pl.BlockSpec((D, D), lambda i: (0, 0)),
            pl.BlockSpec((D,), lambda i: (0,)),
            pl.BlockSpec((D, D), lambda i: (0, 0)),
            pl.BlockSpec((D,), lambda i: (0,)),
        ],
        out_specs=[
            pl.BlockSpec((B, 8, D), lambda i: (0, 0, 0)),
            pl.BlockSpec((B, D), lambda i: (0, 0)),
        ],
        out_shape=[
            jax.ShapeDtypeStruct((B, S, D), jnp.float32),
            jax.ShapeDtypeStruct((B, D), jnp.float32),
        ],
        input_output_aliases={0: 0},
    )(text, graph_ini, W_text, b_text, W_graph, b_graph)

    auf_sc = pl.kernel(
        _sc_copy_body,
        out_type=jax.ShapeDtypeStruct((M, D), jnp.float32),
        mesh=plsc.VectorSubcoreMesh(core_axis_name="c", subcore_axis_name="s"),
        scratch_types=(
            [pltpu.VMEM_SHARED((_CP_CHUNK, 768), jnp.float32)] * _CP_NBUF
            + [pltpu.SemaphoreType.DMA] * (2 * _CP_NBUF)
        ),
    )(all_user_feature)

    auf_full = pl.pallas_call(
        _tc_copy_body,
        grid=(_TC_N,),
        in_specs=[
            pl.BlockSpec(memory_space=pl.ANY),
            pl.BlockSpec((_TC_BLK, D), lambda i: (_TC_OFF + i, 0)),
        ],
        out_specs=pl.BlockSpec((_TC_BLK, D), lambda i: (_TC_OFF + i, 0)),
        out_shape=jax.ShapeDtypeStruct((M, D), jnp.float32),
        input_output_aliases={0: 0},
    )(auf_sc, all_user_feature)

    new_auf = pl.pallas_call(
        _scatter_body,
        in_specs=[
            pl.BlockSpec(memory_space=pltpu.SMEM),
            pl.BlockSpec((B, D), lambda: (0, 0)),
            pl.BlockSpec(memory_space=pl.ANY),
        ],
        out_specs=pl.BlockSpec(memory_space=pl.ANY),
        out_shape=jax.ShapeDtypeStruct((M, D), jnp.float32),
        scratch_shapes=[pltpu.SemaphoreType.DMA],
        input_output_aliases={2: 0},
    )(user_index, graph, auf_full)

    return (text_out, new_auf)


# speedup vs baseline: 1.0834x; 1.0834x over previous
import jax
import jax.numpy as jnp
from jax import lax
from jax.experimental import pallas as pl
from jax.experimental.pallas import tpu as pltpu
from jax.experimental.pallas import tpu_sc as plsc

_NC, _NS = 2, 16
_NW = _NC * _NS

_CP_CHUNK = 336
_CP_NCHUNK = 148
_CP_NBUF = 4
_CP_HALF = 50000
_CP_TAIL = _CP_HALF - _CP_NCHUNK * _CP_CHUNK


def _sc_gather_body(table_hbm, idx_hbm, out_hbm, idx_v, rows_v, sem):
    bpw = idx_v.shape[0]
    wid = lax.axis_index("s") * _NC + lax.axis_index("c")
    base = wid * bpw
    pltpu.sync_copy(idx_hbm.at[pl.ds(base, bpw)], idx_v)
    pltpu.async_copy(table_hbm.at[idx_v], rows_v, sem).wait()
    pltpu.sync_copy(rows_v, out_hbm.at[pl.ds(base, bpw)])


def _sc_copy_body(src_hbm, dst_hbm, buf0, buf1, buf2, buf3,
                  sem_r0, sem_r1, sem_r2, sem_r3,
                  sem_w0, sem_w1, sem_w2, sem_w3):
    core = lax.axis_index("c")
    sid = lax.axis_index("s")
    base = core * _CP_HALF
    bufs = (buf0, buf1, buf2, buf3)
    sems_r = (sem_r0, sem_r1, sem_r2, sem_r3)
    sems_w = (sem_w0, sem_w1, sem_w2, sem_w3)

    def rd(k, p):
        return pltpu.make_async_copy(
            src_hbm.at[pl.ds(base + k * _CP_CHUNK, _CP_CHUNK)], bufs[p],
            sems_r[p])

    def wr(k, p):
        return pltpu.make_async_copy(
            bufs[p], dst_hbm.at[pl.ds(base + k * _CP_CHUNK, _CP_CHUNK)],
            sems_w[p])

    def phased(p, fn):
        for i in range(_CP_NBUF):
            @pl.when(p == i)
            def _(i=i):
                fn(i)

    @pl.when(sid == 0)
    def _():
        for j in range(_CP_NBUF - 1):
            rd(j, j).start()

        def step(k, _):
            def consume(pp):
                rd(k, pp).wait()
                wr(k, pp).start()

            phased(lax.rem(k, _CP_NBUF), consume)

            @pl.when(k + _CP_NBUF - 1 < _CP_NCHUNK)
            def _():
                def prefetch(pp):
                    @pl.when(k >= 1)
                    def _():
                        wr(k - 1, pp).wait()
                    rd(k + _CP_NBUF - 1, pp).start()

                phased(lax.rem(k + _CP_NBUF - 1, _CP_NBUF), prefetch)

            return 0

        lax.fori_loop(0, _CP_NCHUNK, step, 0)
        for k in range(_CP_NCHUNK - _CP_NBUF, _CP_NCHUNK):
            wr(k, k % _CP_NBUF).wait()

        off = base + _CP_NCHUNK * _CP_CHUNK
        tl = pltpu.make_async_copy(
            src_hbm.at[pl.ds(off, _CP_TAIL)],
            buf0.at[pl.ds(0, _CP_TAIL)], sem_r0)
        tl.start()
        tl.wait()
        tw = pltpu.make_async_copy(
            buf0.at[pl.ds(0, _CP_TAIL)],
            dst_hbm.at[pl.ds(off, _CP_TAIL)], sem_w0)
        tw.start()
        tw.wait()


def _mm_body(head_ref, g_ref, wt_ref, bt_ref, wg_ref, bg_ref,
             tok_ref, graph_ref):
    g = g_ref[...]
    t = lax.dot_general(g, wt_ref[...], (((1,), (1,)), ((), ())),
                        preferred_element_type=jnp.float32)
    t = t + bt_ref[...][None, :]
    h = lax.dot_general(g, wg_ref[...], (((1,), (1,)), ((), ())),
                        preferred_element_type=jnp.float32)
    h = h + bg_ref[...][None, :]
    tok_ref[:, 0:1, :] = t[:, None, :]
    tok_ref[:, 1:, :] = head_ref[:, 1:, :]
    graph_ref[...] = h


def _scatter_body(idx_ref, g_ref, cp_ref, out_ref, sem):
    del cp_ref
    n = g_ref.shape[0]

    def row(i):
        return pltpu.make_async_copy(
            g_ref.at[pl.ds(i, 1)],
            out_ref.at[pl.ds(idx_ref[i], 1)],
            sem,
        )

    def fire(i, _):
        row(i).start()
        return 0

    def drain(i, _):
        row(i).wait()
        return 0

    lax.fori_loop(0, n, fire, 0)
    lax.fori_loop(0, n, drain, 0)


def kernel(text, all_user_feature, user_neighbor_index,
           W_text, b_text, W_graph, b_graph):
    B, S, D = text.shape
    M = all_user_feature.shape[0]
    user_index = user_neighbor_index[:, 0]

    bpw = B // _NW
    graph_ini = pl.kernel(
        _sc_gather_body,
        out_type=jax.ShapeDtypeStruct((B, D), jnp.float32),
        mesh=plsc.VectorSubcoreMesh(core_axis_name="c", subcore_axis_name="s"),
        scratch_types=[
            pltpu.VMEM((bpw,), jnp.int32),
            pltpu.VMEM((bpw, D), jnp.float32),
            pltpu.SemaphoreType.DMA,
        ],
    )(all_user_feature, user_index)

    text_out, graph = pl.pallas_call(
        _mm_body,
        grid=(1,),
        in_specs=[
            pl.BlockSpec((B, 8, D), lambda i: (0, 0, 0)),
            pl.BlockSpec((B, D), lambda i: (0, 0)),
            pl.BlockSpec((D, D), lambda i: (0, 0)),
            pl.BlockSpec((D,), lambda i: (0,)),
            pl.BlockSpec((D, D), lambda i: (0, 0)),
            pl.BlockSpec((D,), lambda i: (0,)),
        ],
        out_specs=[
            pl.BlockSpec((B, 8, D), lambda i: (0, 0, 0)),
            pl.BlockSpec((B, D), lambda i: (0, 0)),
        ],
        out_shape=[
            jax.ShapeDtypeStruct((B, S, D), jnp.float32),
            jax.ShapeDtypeStruct((B, D), jnp.float32),
        ],
        input_output_aliases={0: 0},
    )(text, graph_ini, W_text, b_text, W_graph, b_graph)

    auf_copy = pl.kernel(
        _sc_copy_body,
        out_type=jax.ShapeDtypeStruct((M, D), jnp.float32),
        mesh=plsc.VectorSubcoreMesh(core_axis_name="c", subcore_axis_name="s"),
        scratch_types=(
            [pltpu.VMEM_SHARED((_CP_CHUNK, 768), jnp.float32)] * _CP_NBUF
            + [pltpu.SemaphoreType.DMA] * (2 * _CP_NBUF)
        ),
    )(all_user_feature)

    new_auf = pl.pallas_call(
        _scatter_body,
        in_specs=[
            pl.BlockSpec(memory_space=pltpu.SMEM),
            pl.BlockSpec((B, D), lambda: (0, 0)),
            pl.BlockSpec(memory_space=pl.ANY),
        ],
        out_specs=pl.BlockSpec(memory_space=pl.ANY),
        out_shape=jax.ShapeDtypeStruct((M, D), jnp.float32),
        scratch_shapes=[pltpu.SemaphoreType.DMA],
        input_output_aliases={2: 0},
    )(user_index, graph, auf_copy)

    return (text_out, new_auf)
